# baseline (speedup 1.0000x reference)
import functools

import jax
import jax.numpy as jnp
from jax import lax
from jax.experimental import pallas as pl
from jax.experimental.pallas import tpu as pltpu
from jax.experimental.pallas import tpu_sc as plsc

N_CLASSES = 150
EPS = 0.1
ON = 1.0 - EPS
OFF = EPS / (N_CLASSES - 1)

N, H, W = 8, 224, 224
RS = 8
NTASK = N * (H // RS)
CB = 15
NCHUNK = N_CLASSES // CB
NB = 4
NW = 32
TPW = NTASK // NW
L = 16
G = TPW * NCHUNK


def _sc_body(lab_hbm, out_hbm, lab_v, buf_v, sem0, sem1, sem2, sem3):
    sems = (sem0, sem1, sem2, sem3)
    cid = lax.axis_index("c")
    sid = lax.axis_index("s")
    wid = sid * 2 + cid

    on_v = jnp.full((L,), ON, jnp.float32)
    off_v = jnp.full((L,), OFF, jnp.float32)
    lane = lax.iota(jnp.int32, L)

    def fill_c(c, _):
        for p in range(NB):
            for r in range(RS):
                for i in range(W // L):
                    buf_v[p, c, r, pl.ds(i * L, L)] = off_v
        return 0
    lax.fori_loop(0, CB, fill_c, 0)

    def scan_scatter(buf, labq, c0, val_v):
        for r in range(RS):
            for i in range(W // L):
                lab = lab_v[labq, r, pl.ds(i * L, L)]
                c_rel = lab - c0
                mask = c_rel.astype(jnp.uint32) < jnp.uint32(CB)
                plsc.store_scatter(
                    buf, [c_rel, jnp.full((L,), r, jnp.int32),
                          lane + (i * L)], val_v, mask=mask)

    def coords(g):
        ti = g // NCHUNK
        k = g % NCHUNK
        task = wid * TPW + ti
        n = task // (H // RS)
        h0 = (task % (H // RS)) * RS
        return ti, k, n, h0, k * CB

    def chunk_iter(g, _):
        ti, k, n, h0, c0 = coords(g)
        p = g % NB
        buf = buf_v.at[p]

        @pl.when(g >= NB)
        def _wait_and_restore():
            tip, kp, np_, h0p, c0p = coords(g - NB)
            dst = out_hbm.at[np_, pl.ds(c0p, CB), pl.ds(h0p, RS)]
            for q in range(NB):
                @pl.when(p == q)
                def _(q=q):
                    pltpu.make_async_copy(buf, dst, sems[q]).wait()
            scan_scatter(buf, tip % 2, c0p, off_v)

        @pl.when(k == 0)
        def _load_labels():
            pltpu.sync_copy(lab_hbm.at[n, pl.ds(h0, RS)], lab_v.at[ti % 2])

        scan_scatter(buf, ti % 2, c0, on_v)

        dst = out_hbm.at[n, pl.ds(c0, CB), pl.ds(h0, RS)]
        for q in range(NB):
            @pl.when(p == q)
            def _(q=q):
                pltpu.async_copy(buf, dst, sems[q])
        return 0

    lax.fori_loop(0, G, chunk_iter, 0)

    for d in range(NB):
        g = G - NB + d
        ti, k, n, h0, c0 = coords(g)
        pltpu.make_async_copy(
            buf_v.at[g % NB],
            out_hbm.at[n, pl.ds(c0, CB), pl.ds(h0, RS)], sems[g % NB]).wait()


def kernel(labels):
    lab = labels.astype(jnp.int32)
    f = functools.partial(
        pl.kernel,
        mesh=plsc.VectorSubcoreMesh(core_axis_name="c", subcore_axis_name="s"),
        out_type=jax.ShapeDtypeStruct((N, N_CLASSES, H, W), jnp.float32),
        scratch_types=[
            pltpu.VMEM((2, RS, W), jnp.int32),
            pltpu.VMEM((NB, CB, RS, W), jnp.float32),
            pltpu.SemaphoreType.DMA,
            pltpu.SemaphoreType.DMA,
            pltpu.SemaphoreType.DMA,
            pltpu.SemaphoreType.DMA,
        ],
        compiler_params=pltpu.CompilerParams(needs_layout_passes=False),
    )(_sc_body)
    return f(lab)


# speedup vs baseline: 1.2592x; 1.2592x over previous
import functools

import jax
import jax.numpy as jnp
from jax import lax
from jax.experimental import pallas as pl
from jax.experimental.pallas import tpu as pltpu
from jax.experimental.pallas import tpu_sc as plsc

N_CLASSES = 150
EPS = 0.1
ON = 1.0 - EPS
OFF = EPS / (N_CLASSES - 1)

N, H, W = 8, 224, 224
RS = 8
NTASK = N * (H // RS)
CB = 30
NCHUNK = N_CLASSES // CB
NW = 32
TPW = NTASK // NW
L = 16
G = TPW * NCHUNK


def _sc_body(lab_hbm, out_hbm, lab_v, buf_v, sem0, sem1, sem2, sem3):
    semsA = (sem0, sem1)
    semsB = (sem2, sem3)
    cid = lax.axis_index("c")
    sid = lax.axis_index("s")
    wid = sid * 2 + cid

    on_v = jnp.full((L,), ON, jnp.float32)
    off_v = jnp.full((L,), OFF, jnp.float32)
    lane = lax.iota(jnp.int32, L)

    def fill_c(c, _):
        for p in range(2):
            for r in range(RS):
                for i in range(W // L):
                    buf_v[p, c, r, pl.ds(i * L, L)] = off_v
        return 0
    lax.fori_loop(0, CB, fill_c, 0)

    def scan_scatter(buf, labq, c0, val_v):
        for r in range(RS):
            for i in range(W // L):
                lab = lab_v[labq, r, pl.ds(i * L, L)]
                c_rel = lab - c0
                mask = c_rel.astype(jnp.uint32) < jnp.uint32(CB)
                plsc.store_scatter(
                    buf, [c_rel, jnp.full((L,), r, jnp.int32),
                          lane + (i * L)], val_v, mask=mask)

    def coords(g):
        ti = g // NCHUNK
        k = g % NCHUNK
        task = wid * TPW + ti
        n = task // (H // RS)
        h0 = (task % (H // RS)) * RS
        return ti, k, n, h0, k * CB

    WA = 128
    WB = W - WA

    def fire(buf, n, c0, h0, p):
        dstA = out_hbm.at[n, pl.ds(c0, CB), pl.ds(h0, RS), pl.ds(0, WA)]
        dstB = out_hbm.at[n, pl.ds(c0, CB), pl.ds(h0, RS), pl.ds(WA, WB)]
        srcA = buf.at[:, :, pl.ds(0, WA)]
        srcB = buf.at[:, :, pl.ds(WA, WB)]
        for q in range(2):
            @pl.when(p == q)
            def _(q=q):
                pltpu.async_copy(srcA, dstA, semsA[q])
                pltpu.async_copy(srcB, dstB, semsB[q])

    def drain(buf, n, c0, h0, p):
        dstA = out_hbm.at[n, pl.ds(c0, CB), pl.ds(h0, RS), pl.ds(0, WA)]
        dstB = out_hbm.at[n, pl.ds(c0, CB), pl.ds(h0, RS), pl.ds(WA, WB)]
        srcA = buf.at[:, :, pl.ds(0, WA)]
        srcB = buf.at[:, :, pl.ds(WA, WB)]
        for q in range(2):
            @pl.when(p == q)
            def _(q=q):
                pltpu.make_async_copy(srcA, dstA, semsA[q]).wait()
                pltpu.make_async_copy(srcB, dstB, semsB[q]).wait()

    def chunk_iter(g, _):
        ti, k, n, h0, c0 = coords(g)
        p = g % 2
        buf = buf_v.at[p]

        @pl.when(g >= 2)
        def _wait_and_restore():
            tip, kp, np_, h0p, c0p = coords(g - 2)
            drain(buf, np_, c0p, h0p, p)
            scan_scatter(buf, tip % 2, c0p, off_v)

        @pl.when(k == 0)
        def _load_labels():
            pltpu.sync_copy(lab_hbm.at[n, pl.ds(h0, RS)], lab_v.at[ti % 2])

        scan_scatter(buf, ti % 2, c0, on_v)
        fire(buf, n, c0, h0, p)
        return 0

    lax.fori_loop(0, G, chunk_iter, 0)

    for d in range(2):
        g = G - 2 + d
        ti, k, n, h0, c0 = coords(g)
        drain(buf_v.at[g % 2], n, c0, h0, jnp.int32(g % 2))


def kernel(labels):
    lab = labels.astype(jnp.int32)
    f = functools.partial(
        pl.kernel,
        mesh=plsc.VectorSubcoreMesh(core_axis_name="c", subcore_axis_name="s"),
        out_type=jax.ShapeDtypeStruct((N, N_CLASSES, H, W), jnp.float32),
        scratch_types=[
            pltpu.VMEM((2, RS, W), jnp.int32),
            pltpu.VMEM((2, CB, RS, W), jnp.float32),
            pltpu.SemaphoreType.DMA,
            pltpu.SemaphoreType.DMA,
            pltpu.SemaphoreType.DMA,
            pltpu.SemaphoreType.DMA,
        ],
        compiler_params=pltpu.CompilerParams(needs_layout_passes=False),
    )(_sc_body)
    return f(lab)
